# baseline (speedup 1.0000x reference)
import functools

import jax
import jax.numpy as jnp
from jax import lax
from jax.experimental import pallas as pl
from jax.experimental.pallas import tpu as pltpu
from jax.experimental.pallas import tpu_sc as plsc

_D = 64
_SCALE = 8.0
_NC = 2
_NS = 16
_NW = _NC * _NS
_K = 128


@functools.lru_cache(maxsize=None)
def _build(b_total):
    b_per_w = b_total // _NW
    n_chunks = b_per_w // _K
    mesh = plsc.VectorSubcoreMesh(core_axis_name="c", subcore_axis_name="s")

    @functools.partial(
        pl.kernel,
        mesh=mesh,
        out_type=jax.ShapeDtypeStruct((b_total, _D), jnp.float32),
        compiler_params=pltpu.CompilerParams(use_tc_tiling_on_sc=False),
        scratch_types=[
            pltpu.VMEM((b_per_w,), jnp.int32),
            pltpu.VMEM((_K, _D), jnp.float32),
            pltpu.SemaphoreType.DMA,
        ],
    )
    def gather_scale(idx_hbm, table_hbm, out_hbm, idx_v, rows_v, sem):
        wid = lax.axis_index("s") * _NC + lax.axis_index("c")
        base = wid * b_per_w
        pltpu.sync_copy(idx_hbm.at[pl.ds(base, b_per_w)], idx_v)

        def chunk(j, carry):
            row0 = j * _K
            pltpu.async_copy(
                table_hbm.at[idx_v.at[pl.ds(row0, _K)]], rows_v, sem
            ).wait()

            def scale_row(r, c2):
                for u in range(_D // 16):
                    rows_v[r, pl.ds(u * 16, 16)] = (
                        rows_v[r, pl.ds(u * 16, 16)] * _SCALE
                    )
                return c2

            lax.fori_loop(0, _K, scale_row, 0)
            pltpu.sync_copy(rows_v, out_hbm.at[pl.ds(base + row0, _K)])
            return carry

        lax.fori_loop(0, n_chunks, chunk, 0)

    return gather_scale


def kernel(x, embedding):
    b, s = x.shape
    idx = x.reshape(-1).astype(jnp.int32)
    out = _build(b * s)(idx, embedding)
    return out.reshape(b, s, _D)


# speedup vs baseline: 1.1860x; 1.1860x over previous
import functools

import jax
import jax.numpy as jnp
from jax import lax
from jax.experimental import pallas as pl
from jax.experimental.pallas import tpu as pltpu
from jax.experimental.pallas import tpu_sc as plsc

_D = 64
_SCALE = 8.0
_NC = 2
_NS = 16
_NW = _NC * _NS
_K = 128
_NBUF = 4
_PF = 2


@functools.lru_cache(maxsize=None)
def _build(b_total):
    b_per_w = b_total // _NW
    n_chunks = b_per_w // _K
    n_groups = n_chunks // _NBUF
    mesh = plsc.VectorSubcoreMesh(core_axis_name="c", subcore_axis_name="s")

    @functools.partial(
        pl.kernel,
        mesh=mesh,
        out_type=jax.ShapeDtypeStruct((b_total, _D), jnp.float32),
        compiler_params=pltpu.CompilerParams(use_tc_tiling_on_sc=False),
        scratch_types=[
            pltpu.VMEM((b_per_w,), jnp.int32),
            pltpu.VMEM((_NBUF, _K, _D), jnp.float32),
            pltpu.SemaphoreType.DMA((_NBUF,)),
            pltpu.SemaphoreType.DMA((_NBUF,)),
        ],
    )
    def gather_scale(idx_hbm, table_hbm, out_hbm, idx_v, rows_v, g_sem, s_sem):
        wid = lax.axis_index("s") * _NC + lax.axis_index("c")
        base = wid * b_per_w
        pltpu.sync_copy(idx_hbm.at[pl.ds(base, b_per_w)], idx_v)

        def gather_start(chunk, b):
            pltpu.async_copy(
                table_hbm.at[idx_v.at[pl.ds(chunk * _K, _K)]],
                rows_v.at[b],
                g_sem.at[b],
            )

        def gather_wait(b):
            pltpu.make_async_copy(
                table_hbm.at[idx_v.at[pl.ds(0, _K)]], rows_v.at[b], g_sem.at[b]
            ).wait()

        def scatter_start(chunk, b):
            pltpu.async_copy(
                rows_v.at[b], out_hbm.at[pl.ds(base + chunk * _K, _K)], s_sem.at[b]
            )

        def scatter_wait(b):
            pltpu.make_async_copy(
                rows_v.at[b], out_hbm.at[pl.ds(base, _K)], s_sem.at[b]
            ).wait()

        for b in range(_PF):
            gather_start(b, b)

        def group(g, carry):
            for b in range(_NBUF):
                j = g * _NBUF + b
                gather_wait(b)

                def scale_rows(r0, c2):
                    for ur in range(4):
                        r = r0 * 4 + ur
                        for u in range(_D // 16):
                            rows_v[b, r, pl.ds(u * 16, 16)] = (
                                rows_v[b, r, pl.ds(u * 16, 16)] * _SCALE
                            )
                    return c2

                lax.fori_loop(0, _K // 4, scale_rows, 0)
                scatter_start(j, b)

                jp = j + _PF
                bp = (b + _PF) % _NBUF

                @pl.when(jp < n_chunks)
                def _():
                    @pl.when(jp >= _NBUF)
                    def _():
                        scatter_wait(bp)

                    gather_start(jp, bp)

            return carry

        lax.fori_loop(0, n_groups, group, 0)

        for b in range(_NBUF):
            scatter_wait(b)

    return gather_scale


def kernel(x, embedding):
    b, s = x.shape
    idx = x.reshape(-1).astype(jnp.int32)
    out = _build(b * s)(idx, embedding)
    return out.reshape(b, s, _D)
